# baseline (speedup 1.0000x reference)
import functools

import jax
import jax.numpy as jnp
from jax.experimental import pallas as pl


def _vq_kernel(z_ref, cb_ref, cba_ref, c2_ref, zq_ref, idx_ref, vq_ref):
    n = pl.program_id(0)
    b = pl.program_id(1)

    zb = z_ref[0]
    cbn = cb_ref[0]
    cbm2 = cba_ref[0]
    c2 = c2_ref[0]

    xcm2 = jax.lax.dot_general(
        cbm2, zb, (((1,), (0,)), ((), ())),
        preferred_element_type=jnp.float32)
    x2 = jnp.sum(zb * zb, axis=0, keepdims=True)
    scores = (x2 + c2) + xcm2

    idx = jnp.argmin(scores, axis=0)

    onehot = (jax.lax.broadcasted_iota(jnp.int32, scores.shape, 0)
              == idx[None, :]).astype(jnp.float32)
    qT = jax.lax.dot_general(
        cbn, onehot, (((0,), (0,)), ((), ())),
        preferred_element_type=jnp.float32)

    zq_ref[0] = qT
    idx_ref[0, 0, 0] = idx

    @pl.when(jnp.logical_and(n == 0, b == 0))
    def _():
        vq_ref[...] = jnp.zeros((1, 1), jnp.float32)

    diff = zb - qT
    vq_ref[...] += jnp.sum(diff * diff).reshape(1, 1)


@functools.partial(jax.jit, static_argnames=())
def kernel(z, cb):
    beta = 0.25
    B, D, H, W = z.shape
    N, M, ds = cb.shape
    T = B * H * W
    HW = H * W

    zr = z.reshape(B, D, HW)
    c2 = jnp.sum(cb * cb, axis=2, keepdims=True)
    cb_m2 = -2.0 * cb

    zq, idx, vq = pl.pallas_call(
        _vq_kernel,
        grid=(N, B),
        in_specs=[
            pl.BlockSpec((1, ds, HW), lambda n, b: (b, n, 0)),
            pl.BlockSpec((1, M, ds), lambda n, b: (n, 0, 0)),
            pl.BlockSpec((1, M, ds), lambda n, b: (n, 0, 0)),
            pl.BlockSpec((1, M, 1), lambda n, b: (n, 0, 0)),
        ],
        out_specs=[
            pl.BlockSpec((1, ds, HW), lambda n, b: (b, n, 0)),
            pl.BlockSpec((1, 1, 1, HW), lambda n, b: (b, n, 0, 0)),
            pl.BlockSpec((1, 1), lambda n, b: (0, 0)),
        ],
        out_shape=[
            jax.ShapeDtypeStruct((B, D, HW), jnp.float32),
            jax.ShapeDtypeStruct((B, N, 1, HW), jnp.int32),
            jax.ShapeDtypeStruct((1, 1), jnp.float32),
        ],
    )(zr, cb, cb_m2, c2)

    z_q = zq.reshape(B, D, H, W)
    indices = idx.reshape(B, N, H, W)
    vq_loss = (1.0 + beta) * vq[0, 0] / (N * T * ds)
    return (z_q, vq_loss, indices)


# speedup vs baseline: 1.0006x; 1.0006x over previous
import functools

import jax
import jax.numpy as jnp
from jax.experimental import pallas as pl
from jax.experimental.pallas import tpu as pltpu


def _vq_kernel(z_ref, cb_ref, cba_ref, c2_ref, zq_ref, idx_ref, vq_ref):
    zb = z_ref[0]
    cbn = cb_ref[0]
    cbm2 = cba_ref[0]
    c2 = c2_ref[0]

    xcm2 = jax.lax.dot_general(
        cbm2, zb, (((1,), (0,)), ((), ())),
        preferred_element_type=jnp.float32)
    x2 = jnp.sum(zb * zb, axis=0, keepdims=True)
    scores = (x2 + c2) + xcm2

    idx = jnp.argmin(scores, axis=0)

    onehot = (jax.lax.broadcasted_iota(jnp.int32, scores.shape, 0)
              == idx[None, :]).astype(jnp.float32)
    qT = jax.lax.dot_general(
        cbn, onehot, (((0,), (0,)), ((), ())),
        preferred_element_type=jnp.float32)

    zq_ref[0] = qT
    idx_ref[0, 0, 0] = idx

    diff = zb - qT
    vq_ref[...] = jnp.sum(diff * diff).reshape(1, 1, 1, 1)


@functools.partial(jax.jit, static_argnames=())
def kernel(z, cb):
    beta = 0.25
    B, D, H, W = z.shape
    N, M, ds = cb.shape
    T = B * H * W
    HW = H * W

    zr = z.reshape(B, D, HW)
    c2 = jnp.sum(cb * cb, axis=2, keepdims=True)
    cb_m2 = -2.0 * cb

    zq, idx, vq = pl.pallas_call(
        _vq_kernel,
        grid=(N, B),
        in_specs=[
            pl.BlockSpec((1, ds, HW), lambda n, b: (b, n, 0)),
            pl.BlockSpec((1, M, ds), lambda n, b: (n, 0, 0)),
            pl.BlockSpec((1, M, ds), lambda n, b: (n, 0, 0)),
            pl.BlockSpec((1, M, 1), lambda n, b: (n, 0, 0)),
        ],
        out_specs=[
            pl.BlockSpec((1, ds, HW), lambda n, b: (b, n, 0)),
            pl.BlockSpec((1, 1, 1, HW), lambda n, b: (b, n, 0, 0)),
            pl.BlockSpec((1, 1, 1, 1), lambda n, b: (n, b, 0, 0)),
        ],
        out_shape=[
            jax.ShapeDtypeStruct((B, D, HW), jnp.float32),
            jax.ShapeDtypeStruct((B, N, 1, HW), jnp.int32),
            jax.ShapeDtypeStruct((N, B, 1, 1), jnp.float32),
        ],
        compiler_params=pltpu.CompilerParams(
            dimension_semantics=("parallel", "parallel")),
    )(zr, cb, cb_m2, c2)

    z_q = zq.reshape(B, D, H, W)
    indices = idx.reshape(B, N, H, W)
    vq_loss = (1.0 + beta) * jnp.sum(vq) / (N * T * ds)
    return (z_q, vq_loss, indices)
